# baseline (speedup 1.0000x reference)
import jax
import jax.numpy as jnp
from jax import lax
from jax.experimental import pallas as pl
from jax.experimental.pallas import tpu as pltpu

_G = 64
_TN = 2048
_TR = 2048
_VMEM_LIMIT = 60 * 1024 * 1024


def _mlp_gate(c, wc_ref, bc_ref, k0):
    hd = wc_ref.shape[1]
    w = lambda k: wc_ref[(k0 + k) * hd:(k0 + k + 1) * hd, :]
    b = lambda k: bc_ref[k0 + k:k0 + k + 1, :]
    z = jnp.dot(c.astype(jnp.bfloat16), w(0),
                preferred_element_type=jnp.float32) + b(0)
    z = jnp.maximum(z, 0.0)
    z = jnp.dot(z.astype(jnp.bfloat16), w(1),
                preferred_element_type=jnp.float32) + b(1)
    z = jnp.maximum(z, 0.0)
    z = jnp.dot(z.astype(jnp.bfloat16), w(2),
                preferred_element_type=jnp.float32) + b(2)
    return jax.nn.sigmoid(z)


def _gather_rows(gid, gate_ref):
    tn = gid.shape[1]
    onehot = (gid == lax.broadcasted_iota(jnp.int32, (_G, tn), 0)
              ).astype(jnp.bfloat16)
    return lax.dot_general(onehot, gate_ref[...],
                           (((0,), (0,)), ((), ())),
                           preferred_element_type=jnp.float32)


def _edge_gids(src_ref, b2_ref):
    tn = src_ref.shape[-1]
    src = src_ref[0:1, :]
    hi = src >> 7
    lo = src & 127
    n_rows = b2_ref.shape[0]
    oh_hi = (hi == lax.broadcasted_iota(jnp.int32, (n_rows, tn), 0)
             ).astype(jnp.bfloat16)
    m1t = lax.dot_general(b2_ref[...].astype(jnp.bfloat16), oh_hi,
                          (((0,), (0,)), ((), ())),
                          preferred_element_type=jnp.float32)
    oh_lo = (lo == lax.broadcasted_iota(jnp.int32, (128, tn), 0)
             ).astype(jnp.float32)
    gid = jnp.sum(m1t * oh_lo, axis=0, keepdims=True)
    return gid.astype(jnp.int32)


def _reduce_body(gid0_ref, gid1_ref, h0_ref, h1_ref, part_ref):
    hdim = h0_ref.shape[1]
    sums = jnp.zeros((_G, hdim), jnp.float32)
    cnts = jnp.zeros((_G, 1), jnp.float32)
    for gid_ref, h_ref in ((gid0_ref, h0_ref), (gid1_ref, h1_ref)):
        tn = gid_ref.shape[1]
        onehot = gid_ref[...] == lax.broadcasted_iota(jnp.int32, (_G, tn), 0)
        sums = sums + jnp.dot(onehot.astype(jnp.bfloat16),
                              h_ref[...].astype(jnp.bfloat16),
                              preferred_element_type=jnp.float32)
        cnts = cnts + jnp.sum(onehot.astype(jnp.float32), axis=1,
                              keepdims=True)
    part_ref[0, :, :hdim] = sums
    part_ref[0, :, hdim:] = jnp.broadcast_to(cnts, (_G, 128))


def _reduce_partials(gid2d, h):
    n, hdim = h.shape
    tr = n // 4
    return pl.pallas_call(
        _reduce_body,
        out_shape=jax.ShapeDtypeStruct((2, _G, hdim + 128), jnp.float32),
        grid_spec=pltpu.PrefetchScalarGridSpec(
            num_scalar_prefetch=0,
            grid=(2,),
            in_specs=[pl.BlockSpec((1, tr), lambda c: (0, 2 * c)),
                      pl.BlockSpec((1, tr), lambda c: (0, 2 * c + 1)),
                      pl.BlockSpec((tr, hdim), lambda c: (2 * c, 0)),
                      pl.BlockSpec((tr, hdim), lambda c: (2 * c + 1, 0))],
            out_specs=pl.BlockSpec((1, _G, hdim + 128), lambda c: (c, 0, 0)),
        ),
        compiler_params=pltpu.CompilerParams(
            dimension_semantics=("parallel",),
            vmem_limit_bytes=_VMEM_LIMIT),
    )(gid2d, gid2d, h, h)


def _make_apply_body(nv_c):
    def _body(part_ref, wc_ref, bc_ref, gidv_ref, xv_ref, src_ref, b2_ref,
              xe_ref, ov_ref, oe_ref, gv_s, ge_s):
        j = pl.program_id(1)

        @pl.when(j == 0)
        def _gates():
            hdim = gv_s.shape[1]
            acc = part_ref[0] + part_ref[1]
            c = acc[:, :hdim] / jnp.maximum(acc[:, hdim:hdim + 1], 1.0)
            gv = _mlp_gate(c, wc_ref, bc_ref, 0)
            gv_s[...] = gv.astype(jnp.bfloat16)
            ge_s[...] = _mlp_gate(c * gv, wc_ref, bc_ref, 3).astype(jnp.bfloat16)

        @pl.when(j < nv_c)
        def _node_phase():
            ov_ref[...] = xv_ref[...] * _gather_rows(gidv_ref[...], gv_s)

        @pl.when(j >= nv_c)
        def _edge_phase():
            gid_e = _edge_gids(src_ref, b2_ref)
            oe_ref[...] = xe_ref[...] * _gather_rows(gid_e, ge_s)

    return _body


def _apply_gates(parts, wcat, bcat, gidv, x_v, eidx, b2, x_e):
    nv, hdim = x_v.shape
    ne = x_e.shape[0]
    tn = min(_TN, nv // 2, ne // 2)
    nv_c = nv // tn // 2
    ne_c = ne // tn // 2
    vi = lambda c, j: c * nv_c + jnp.minimum(j, nv_c - 1)
    ei = lambda c, j: c * ne_c + jnp.maximum(j - nv_c, 0)
    inv = lambda c, j: (0, 0)

    return pl.pallas_call(
        _make_apply_body(nv_c),
        out_shape=(jax.ShapeDtypeStruct((nv, hdim), x_v.dtype),
                   jax.ShapeDtypeStruct((ne, hdim), x_e.dtype)),
        grid_spec=pltpu.PrefetchScalarGridSpec(
            num_scalar_prefetch=0,
            grid=(2, nv_c + ne_c),
            in_specs=[pl.BlockSpec(parts.shape, lambda c, j: (0, 0, 0)),
                      pl.BlockSpec(wcat.shape, inv),
                      pl.BlockSpec(bcat.shape, inv),
                      pl.BlockSpec((1, tn), lambda c, j: (0, vi(c, j))),
                      pl.BlockSpec((tn, hdim), lambda c, j: (vi(c, j), 0)),
                      pl.BlockSpec((2, tn), lambda c, j: (0, ei(c, j))),
                      pl.BlockSpec(b2.shape, inv),
                      pl.BlockSpec((tn, hdim), lambda c, j: (ei(c, j), 0))],
            out_specs=[pl.BlockSpec((tn, hdim), lambda c, j: (vi(c, j), 0)),
                       pl.BlockSpec((tn, hdim), lambda c, j: (ei(c, j), 0))],
            scratch_shapes=[pltpu.VMEM((_G, hdim), jnp.bfloat16),
                            pltpu.VMEM((_G, hdim), jnp.bfloat16)],
        ),
        compiler_params=pltpu.CompilerParams(
            dimension_semantics=("parallel", "arbitrary"),
            vmem_limit_bytes=_VMEM_LIMIT),
    )(parts, wcat, bcat, gidv, x_v, eidx, b2, x_e)


def kernel(h_V, h_E, edge_idx, batch_id,
           v_w1, v_b1, v_w2, v_b2, v_w3, v_b3,
           e_w1, e_b1, e_w2, e_b2, e_w3, e_b3):
    n, _ = h_V.shape
    gid_v = batch_id.astype(jnp.int32).reshape(1, n)
    wcat = jnp.concatenate([v_w1, v_w2, v_w3, e_w1, e_w2, e_w3],
                           axis=0).astype(jnp.bfloat16)
    bcat = jnp.concatenate([v_b1, v_b2, v_b3, e_b1, e_b2, e_b3], axis=0)
    parts = _reduce_partials(gid_v, h_V)
    b2 = batch_id.astype(jnp.int32).reshape(n // 128, 128)
    out_V, out_E = _apply_gates(parts, wcat, bcat, gid_v, h_V,
                                edge_idx.astype(jnp.int32), b2, h_E)
    return out_V, out_E


# speedup vs baseline: 1.0931x; 1.0931x over previous
import jax
import jax.numpy as jnp
from jax import lax
from jax.experimental import pallas as pl
from jax.experimental.pallas import tpu as pltpu

_G = 64
_TN = 2048
_TR = 2048
_VMEM_LIMIT = 48 * 1024 * 1024


def _mlp_gate(c, wc_ref, bc_ref, k0):
    hd = wc_ref.shape[1]
    w = lambda k: wc_ref[(k0 + k) * hd:(k0 + k + 1) * hd, :]
    b = lambda k: bc_ref[k0 + k:k0 + k + 1, :]
    z = jnp.dot(c.astype(jnp.bfloat16), w(0),
                preferred_element_type=jnp.float32) + b(0)
    z = jnp.maximum(z, 0.0)
    z = jnp.dot(z.astype(jnp.bfloat16), w(1),
                preferred_element_type=jnp.float32) + b(1)
    z = jnp.maximum(z, 0.0)
    z = jnp.dot(z.astype(jnp.bfloat16), w(2),
                preferred_element_type=jnp.float32) + b(2)
    return jax.nn.sigmoid(z)


def _gather_rows(gid, gate_ref):
    tn = gid.shape[1]
    onehot = (gid == lax.broadcasted_iota(jnp.int32, (_G, tn), 0)
              ).astype(jnp.bfloat16)
    return lax.dot_general(onehot, gate_ref[...],
                           (((0,), (0,)), ((), ())),
                           preferred_element_type=jnp.float32)


def _edge_gids(src_ref, b2_ref):
    tn = src_ref.shape[-1]
    src = src_ref[0:1, :]
    hi = src >> 7
    lo = src & 127
    n_rows = b2_ref.shape[0]
    oh_hi = (hi == lax.broadcasted_iota(jnp.int32, (n_rows, tn), 0)
             ).astype(jnp.bfloat16)
    m1t = lax.dot_general(b2_ref[...].astype(jnp.bfloat16), oh_hi,
                          (((0,), (0,)), ((), ())),
                          preferred_element_type=jnp.float32)
    oh_lo = (lo == lax.broadcasted_iota(jnp.int32, (128, tn), 0)
             ).astype(jnp.float32)
    gid = jnp.sum(m1t * oh_lo, axis=0, keepdims=True)
    return gid.astype(jnp.int32)


def _reduce_body(gid0_ref, gid1_ref, h0_ref, h1_ref, part_ref):
    hdim = h0_ref.shape[1]
    sums = jnp.zeros((_G, hdim), jnp.float32)
    cnts = jnp.zeros((_G, 1), jnp.float32)
    for gid_ref, h_ref in ((gid0_ref, h0_ref), (gid1_ref, h1_ref)):
        tn = gid_ref.shape[1]
        onehot = gid_ref[...] == lax.broadcasted_iota(jnp.int32, (_G, tn), 0)
        sums = sums + jnp.dot(onehot.astype(jnp.bfloat16),
                              h_ref[...].astype(jnp.bfloat16),
                              preferred_element_type=jnp.float32)
        cnts = cnts + jnp.sum(onehot.astype(jnp.float32), axis=1,
                              keepdims=True)
    part_ref[0, :, :hdim] = sums
    part_ref[0, :, hdim:] = jnp.broadcast_to(cnts, (_G, 128))


def _reduce_partials(gid2d, h):
    n, hdim = h.shape
    tr = n // 4
    return pl.pallas_call(
        _reduce_body,
        out_shape=jax.ShapeDtypeStruct((2, _G, hdim + 128), jnp.float32),
        grid_spec=pltpu.PrefetchScalarGridSpec(
            num_scalar_prefetch=0,
            grid=(2,),
            in_specs=[pl.BlockSpec((1, tr), lambda c: (0, 2 * c)),
                      pl.BlockSpec((1, tr), lambda c: (0, 2 * c + 1)),
                      pl.BlockSpec((tr, hdim), lambda c: (2 * c, 0)),
                      pl.BlockSpec((tr, hdim), lambda c: (2 * c + 1, 0))],
            out_specs=pl.BlockSpec((1, _G, hdim + 128), lambda c: (c, 0, 0)),
        ),
        compiler_params=pltpu.CompilerParams(
            dimension_semantics=("parallel",),
            vmem_limit_bytes=_VMEM_LIMIT),
    )(gid2d, gid2d, h, h)


def _make_apply_body(nv_c):
    def _body(part_ref, wc_ref, bc_ref, gidv_ref, xv_ref, src_ref, b2_ref,
              xe_ref, ov_ref, oe_ref, gv_s, ge_s):
        j = pl.program_id(1)

        @pl.when(j == 0)
        def _gates():
            hdim = gv_s.shape[1]
            acc = part_ref[0] + part_ref[1]
            c = acc[:, :hdim] / jnp.maximum(acc[:, hdim:hdim + 1], 1.0)
            gv = _mlp_gate(c, wc_ref, bc_ref, 0)
            gv_s[...] = gv.astype(jnp.bfloat16)
            ge_s[...] = _mlp_gate(c * gv, wc_ref, bc_ref, 3).astype(jnp.bfloat16)

        @pl.when(j < nv_c)
        def _node_phase():
            ov_ref[...] = xv_ref[...] * _gather_rows(gidv_ref[...], gv_s)

        @pl.when(j >= nv_c)
        def _edge_phase():
            gid_e = _edge_gids(src_ref, b2_ref)
            oe_ref[...] = xe_ref[...] * _gather_rows(gid_e, ge_s)

    return _body


def _apply_gates(parts, wcat, bcat, gidv, x_v, eidx, b2, x_e):
    nv, hdim = x_v.shape
    ne = x_e.shape[0]
    tn = min(_TN, nv // 2, ne // 2)
    nv_c = nv // tn // 2
    ne_c = ne // tn // 2
    vi = lambda c, j: c * nv_c + jnp.minimum(j, nv_c - 1)
    ei = lambda c, j: c * ne_c + jnp.maximum(j - nv_c, 0)
    inv = lambda c, j: (0, 0)

    return pl.pallas_call(
        _make_apply_body(nv_c),
        out_shape=(jax.ShapeDtypeStruct((nv, hdim), x_v.dtype),
                   jax.ShapeDtypeStruct((ne, hdim), x_e.dtype)),
        grid_spec=pltpu.PrefetchScalarGridSpec(
            num_scalar_prefetch=0,
            grid=(2, nv_c + ne_c),
            in_specs=[pl.BlockSpec(parts.shape, lambda c, j: (0, 0, 0)),
                      pl.BlockSpec(wcat.shape, inv),
                      pl.BlockSpec(bcat.shape, inv),
                      pl.BlockSpec((1, tn), lambda c, j: (0, vi(c, j))),
                      pl.BlockSpec((tn, hdim), lambda c, j: (vi(c, j), 0)),
                      pl.BlockSpec((2, tn), lambda c, j: (0, ei(c, j))),
                      pl.BlockSpec(b2.shape, inv),
                      pl.BlockSpec((tn, hdim), lambda c, j: (ei(c, j), 0))],
            out_specs=[pl.BlockSpec((tn, hdim), lambda c, j: (vi(c, j), 0)),
                       pl.BlockSpec((tn, hdim), lambda c, j: (ei(c, j), 0))],
            scratch_shapes=[pltpu.VMEM((_G, hdim), jnp.bfloat16),
                            pltpu.VMEM((_G, hdim), jnp.bfloat16)],
        ),
        compiler_params=pltpu.CompilerParams(
            dimension_semantics=("parallel", "arbitrary"),
            vmem_limit_bytes=_VMEM_LIMIT),
    )(parts, wcat, bcat, gidv, x_v, eidx, b2, x_e)


def kernel(h_V, h_E, edge_idx, batch_id,
           v_w1, v_b1, v_w2, v_b2, v_w3, v_b3,
           e_w1, e_b1, e_w2, e_b2, e_w3, e_b3):
    n, _ = h_V.shape
    gid_v = batch_id.astype(jnp.int32).reshape(1, n)
    wcat = jnp.concatenate([v_w1, v_w2, v_w3, e_w1, e_w2, e_w3],
                           axis=0).astype(jnp.bfloat16)
    bcat = jnp.concatenate([v_b1, v_b2, v_b3, e_b1, e_b2, e_b3], axis=0)
    parts = _reduce_partials(gid_v, h_V)
    b2 = batch_id.astype(jnp.int32).reshape(n // 128, 128)
    out_V, out_E = _apply_gates(parts, wcat, bcat, gid_v, h_V,
                                edge_idx.astype(jnp.int32), b2, h_E)
    return out_V, out_E


# speedup vs baseline: 1.1065x; 1.0122x over previous
import jax
import jax.numpy as jnp
from jax import lax
from jax.experimental import pallas as pl
from jax.experimental.pallas import tpu as pltpu

_G = 64
_TN = 2048
_TR = 2048
_VMEM_LIMIT = 48 * 1024 * 1024


def _mlp_gate(c, wc_ref, bc_ref, k0):
    hd = wc_ref.shape[1]
    w = lambda k: wc_ref[(k0 + k) * hd:(k0 + k + 1) * hd, :]
    b = lambda k: bc_ref[k0 + k:k0 + k + 1, :]
    z = jnp.dot(c.astype(jnp.bfloat16), w(0),
                preferred_element_type=jnp.float32) + b(0)
    z = jnp.maximum(z, 0.0)
    z = jnp.dot(z.astype(jnp.bfloat16), w(1),
                preferred_element_type=jnp.float32) + b(1)
    z = jnp.maximum(z, 0.0)
    z = jnp.dot(z.astype(jnp.bfloat16), w(2),
                preferred_element_type=jnp.float32) + b(2)
    return jax.nn.sigmoid(z)


def _gather_rows(gid, gate_ref):
    tn = gid.shape[1]
    onehot = (gid == lax.broadcasted_iota(jnp.int32, (_G, tn), 0)
              ).astype(jnp.bfloat16)
    return lax.dot_general(onehot, gate_ref[...],
                           (((0,), (0,)), ((), ())),
                           preferred_element_type=jnp.float32)


def _edge_gids(src_ref, b2_ref):
    tn = src_ref.shape[-1]
    src = src_ref[0:1, :]
    hi = src >> 7
    lo = src & 127
    n_rows = b2_ref.shape[0]
    oh_hi = (hi == lax.broadcasted_iota(jnp.int32, (n_rows, tn), 0)
             ).astype(jnp.bfloat16)
    m1t = lax.dot_general(b2_ref[...].astype(jnp.bfloat16), oh_hi,
                          (((0,), (0,)), ((), ())),
                          preferred_element_type=jnp.float32)
    oh_lo = (lo == lax.broadcasted_iota(jnp.int32, (128, tn), 0)
             ).astype(jnp.float32)
    gid = jnp.sum(m1t * oh_lo, axis=0, keepdims=True)
    return gid.astype(jnp.int32)


def _reduce_body(gid0_ref, gid1_ref, gid2_ref, gid3_ref,
                 h0_ref, h1_ref, h2_ref, h3_ref, part_ref):
    hdim = h0_ref.shape[1]
    sums = jnp.zeros((_G, hdim), jnp.float32)
    cnts = jnp.zeros((_G, 1), jnp.float32)
    for gid_ref, h_ref in ((gid0_ref, h0_ref), (gid1_ref, h1_ref),
                           (gid2_ref, h2_ref), (gid3_ref, h3_ref)):
        tn = gid_ref.shape[1]
        onehot = gid_ref[...] == lax.broadcasted_iota(jnp.int32, (_G, tn), 0)
        sums = sums + jnp.dot(onehot.astype(jnp.bfloat16),
                              h_ref[...].astype(jnp.bfloat16),
                              preferred_element_type=jnp.float32)
        cnts = cnts + jnp.sum(onehot.astype(jnp.float32), axis=1,
                              keepdims=True)
    part_ref[0, :, :hdim] = sums
    part_ref[0, :, hdim:] = jnp.broadcast_to(cnts, (_G, 128))


def _reduce_partials(gid2d, h):
    n, hdim = h.shape
    tr = n // 8
    return pl.pallas_call(
        _reduce_body,
        out_shape=jax.ShapeDtypeStruct((2, _G, hdim + 128), jnp.float32),
        grid_spec=pltpu.PrefetchScalarGridSpec(
            num_scalar_prefetch=0,
            grid=(2,),
            in_specs=[pl.BlockSpec((1, tr), lambda c: (0, 4 * c)),
                      pl.BlockSpec((1, tr), lambda c: (0, 4 * c + 1)),
                      pl.BlockSpec((1, tr), lambda c: (0, 4 * c + 2)),
                      pl.BlockSpec((1, tr), lambda c: (0, 4 * c + 3)),
                      pl.BlockSpec((tr, hdim), lambda c: (4 * c, 0)),
                      pl.BlockSpec((tr, hdim), lambda c: (4 * c + 1, 0)),
                      pl.BlockSpec((tr, hdim), lambda c: (4 * c + 2, 0)),
                      pl.BlockSpec((tr, hdim), lambda c: (4 * c + 3, 0))],
            out_specs=pl.BlockSpec((1, _G, hdim + 128), lambda c: (c, 0, 0)),
        ),
        compiler_params=pltpu.CompilerParams(
            dimension_semantics=("parallel",),
            vmem_limit_bytes=_VMEM_LIMIT),
    )(gid2d, gid2d, gid2d, gid2d, h, h, h, h)


def _make_apply_body(nv_c):
    def _body(part_ref, wc_ref, bc_ref, gidv_ref, xv_ref, src_ref, b2_ref,
              xe_ref, ov_ref, oe_ref, gv_s, ge_s):
        j = pl.program_id(1)

        @pl.when(j == 0)
        def _gates():
            hdim = gv_s.shape[1]
            acc = part_ref[0] + part_ref[1]
            c = acc[:, :hdim] / jnp.maximum(acc[:, hdim:hdim + 1], 1.0)
            gv = _mlp_gate(c, wc_ref, bc_ref, 0)
            gv_s[...] = gv.astype(jnp.bfloat16)
            ge_s[...] = _mlp_gate(c * gv, wc_ref, bc_ref, 3).astype(jnp.bfloat16)

        @pl.when(j < nv_c)
        def _node_phase():
            ov_ref[...] = xv_ref[...] * _gather_rows(gidv_ref[...], gv_s)

        @pl.when(j >= nv_c)
        def _edge_phase():
            gid_e = _edge_gids(src_ref, b2_ref)
            oe_ref[...] = xe_ref[...] * _gather_rows(gid_e, ge_s)

    return _body


def _apply_gates(parts, wcat, bcat, gidv, x_v, eidx, b2, x_e):
    nv, hdim = x_v.shape
    ne = x_e.shape[0]
    tn = min(_TN, nv // 2, ne // 2)
    nv_c = nv // tn // 2
    ne_c = ne // tn // 2
    vi = lambda c, j: c * nv_c + jnp.minimum(j, nv_c - 1)
    ei = lambda c, j: c * ne_c + jnp.maximum(j - nv_c, 0)
    inv = lambda c, j: (0, 0)

    return pl.pallas_call(
        _make_apply_body(nv_c),
        out_shape=(jax.ShapeDtypeStruct((nv, hdim), x_v.dtype),
                   jax.ShapeDtypeStruct((ne, hdim), x_e.dtype)),
        grid_spec=pltpu.PrefetchScalarGridSpec(
            num_scalar_prefetch=0,
            grid=(2, nv_c + ne_c),
            in_specs=[pl.BlockSpec(parts.shape, lambda c, j: (0, 0, 0)),
                      pl.BlockSpec(wcat.shape, inv),
                      pl.BlockSpec(bcat.shape, inv),
                      pl.BlockSpec((1, tn), lambda c, j: (0, vi(c, j))),
                      pl.BlockSpec((tn, hdim), lambda c, j: (vi(c, j), 0)),
                      pl.BlockSpec((2, tn), lambda c, j: (0, ei(c, j))),
                      pl.BlockSpec(b2.shape, inv),
                      pl.BlockSpec((tn, hdim), lambda c, j: (ei(c, j), 0))],
            out_specs=[pl.BlockSpec((tn, hdim), lambda c, j: (vi(c, j), 0)),
                       pl.BlockSpec((tn, hdim), lambda c, j: (ei(c, j), 0))],
            scratch_shapes=[pltpu.VMEM((_G, hdim), jnp.bfloat16),
                            pltpu.VMEM((_G, hdim), jnp.bfloat16)],
        ),
        compiler_params=pltpu.CompilerParams(
            dimension_semantics=("parallel", "arbitrary"),
            vmem_limit_bytes=_VMEM_LIMIT),
    )(parts, wcat, bcat, gidv, x_v, eidx, b2, x_e)


def kernel(h_V, h_E, edge_idx, batch_id,
           v_w1, v_b1, v_w2, v_b2, v_w3, v_b3,
           e_w1, e_b1, e_w2, e_b2, e_w3, e_b3):
    n, _ = h_V.shape
    gid_v = batch_id.astype(jnp.int32).reshape(1, n)
    wcat = jnp.concatenate([v_w1, v_w2, v_w3, e_w1, e_w2, e_w3],
                           axis=0).astype(jnp.bfloat16)
    bcat = jnp.concatenate([v_b1, v_b2, v_b3, e_b1, e_b2, e_b3], axis=0)
    parts = _reduce_partials(gid_v, h_V)
    b2 = batch_id.astype(jnp.int32).reshape(n // 128, 128)
    out_V, out_E = _apply_gates(parts, wcat, bcat, gid_v, h_V,
                                edge_idx.astype(jnp.int32), b2, h_E)
    return out_V, out_E
